# baseline (speedup 1.0000x reference)
import functools

import jax
import jax.numpy as jnp
from jax import lax
from jax.experimental import pallas as pl
from jax.experimental.pallas import tpu as pltpu
from jax.experimental.pallas import tpu_sc as plsc

_INFO = plsc.get_sparse_core_info()
_NC = _INFO.num_cores
_NS = _INFO.num_subcores
_NW = _NC * _NS
_LANES = _INFO.num_lanes

_REP = 64


def kernel(inputs, embedding):
    batch = inputs.shape[0]
    d = embedding.shape[1]
    rows_w = batch // _NW
    n_out = rows_w // _REP

    mesh = plsc.VectorSubcoreMesh(core_axis_name="c", subcore_axis_name="s")

    @functools.partial(
        pl.kernel,
        mesh=mesh,
        out_type=jax.ShapeDtypeStruct((batch, d), embedding.dtype),
        scratch_types=[
            pltpu.VMEM((_REP, d), jnp.float32),
            pltpu.SemaphoreType.DMA,
        ],
    )
    def sc_broadcast(emb_hbm, out_hbm, buf, sem):
        wid = lax.axis_index("s") * _NC + lax.axis_index("c")
        base = wid * rows_w
        pltpu.sync_copy(emb_hbm, buf.at[pl.ds(0, 1)])
        row = [buf[0, pl.ds(_LANES * j, _LANES)] for j in range(d // _LANES)]
        for r in range(1, _REP):
            for j in range(d // _LANES):
                buf[r, pl.ds(_LANES * j, _LANES)] = row[j]
        copies = [
            pltpu.async_copy(buf, out_hbm.at[pl.ds(base + c * _REP, _REP)], sem)
            for c in range(n_out)
        ]
        for cp in copies:
            cp.wait()

    return sc_broadcast(embedding)


# speedup vs baseline: 1.0245x; 1.0245x over previous
import functools

import jax
import jax.numpy as jnp
from jax import lax
from jax.experimental import pallas as pl
from jax.experimental.pallas import tpu as pltpu
from jax.experimental.pallas import tpu_sc as plsc

_INFO = plsc.get_sparse_core_info()
_NC = _INFO.num_cores
_NS = _INFO.num_subcores
_NW = _NC * _NS
_LANES = _INFO.num_lanes

_REP = 64


def kernel(inputs, embedding):
    batch = inputs.shape[0]
    d = embedding.shape[1]
    rows_w = batch // _NW
    n_out = rows_w // _REP

    mesh = plsc.VectorSubcoreMesh(core_axis_name="c", subcore_axis_name="s")

    @functools.partial(
        pl.kernel,
        mesh=mesh,
        out_type=jax.ShapeDtypeStruct((batch, d), embedding.dtype),
        scratch_types=[
            pltpu.VMEM((_REP, d), jnp.float32),
            pltpu.SemaphoreType.DMA,
        ],
    )
    def sc_broadcast(emb_hbm, out_hbm, buf, sem):
        wid = lax.axis_index("s") * _NC + lax.axis_index("c")
        base = wid * rows_w
        pltpu.sync_copy(emb_hbm, buf.at[pl.ds(0, 1)])
        row = [buf[0, pl.ds(_LANES * j, _LANES)] for j in range(d // _LANES)]

        def _rep(r, carry):
            for j in range(d // _LANES):
                buf[r, pl.ds(_LANES * j, _LANES)] = row[j]
            return carry

        lax.fori_loop(1, _REP, _rep, 0, unroll=False)
        copies = [
            pltpu.async_copy(buf, out_hbm.at[pl.ds(base + c * _REP, _REP)], sem)
            for c in range(n_out)
        ]
        for cp in copies:
            cp.wait()

    return sc_broadcast(embedding)
